# Initial kernel scaffold; baseline (speedup 1.0000x reference)
import jax
import jax.numpy as jnp
from jax.experimental import pallas as pl


def kernel(x, edge_index, edge_attr, Ws_t0, bs_t0, Wk_t0_k1, bk_t0_k1, Ws_t1, bs_t1, Wk_t1_k1, bk_t1_k1, Wk_t1_k2, bk_t1_k2, Ws_t2, bs_t2, Wk_t2_k1, bk_t2_k1, Wk_t2_k2, bk_t2_k2, Wk_t2_k3, bk_t2_k3, Whead, bhead):
    raise NotImplementedError("write your pallas kernel here")


# speedup vs baseline: 2.1458x; 2.1458x over previous
import functools

import jax
import jax.numpy as jnp
from jax import lax
from jax.experimental import pallas as pl
from jax.experimental.pallas import tpu as pltpu
from jax.experimental.pallas import tpu_sc as plsc

_N = 10000
_D = 128
_NC = 2
_NS = 16
_CH = 128
_NACC = 10112
_RPW = _NACC // _NS


@functools.lru_cache(maxsize=None)
def _seg_sum(k, e_pad):
    cpw = e_pad // (_NC * _NS * _CH)
    mesh = plsc.VectorSubcoreMesh(core_axis_name="c", subcore_axis_name="s")

    def body(xt, src, dst, attr, zeros, out, srcv, dstv, attrv, dstm, rows,
             acc, sem):
        cid = lax.axis_index("c")
        sid = lax.axis_index("s")
        wid = cid * _NS + sid
        r0 = sid * _RPW
        pltpu.sync_copy(zeros.at[pl.ds(r0, _RPW)], acc.at[pl.ds(r0, _RPW)])
        plsc.subcore_barrier()
        base0 = wid * cpw * _CH
        trash = _N + sid

        def chunk(i, carry):
            base = base0 + i * _CH
            pltpu.sync_copy(src.at[pl.ds(base, _CH)], srcv)
            pltpu.sync_copy(dst.at[pl.ds(base, _CH)], dstv)
            pltpu.sync_copy(attr.at[pl.ds(base, _CH)], attrv)
            pltpu.async_copy(xt.at[srcv], rows, sem).wait()
            for j in range(_CH // 16):
                sl = pl.ds(j * 16, 16)
                dstm[sl] = jnp.where(attrv[sl] == k, dstv[sl], trash)
            pltpu.sync_copy(rows, acc.at[dstm], add=True)
            return carry

        lax.fori_loop(0, cpw, chunk, 0)
        plsc.subcore_barrier()
        pltpu.sync_copy(acc.at[pl.ds(r0, _RPW)], out.at[cid, pl.ds(r0, _RPW)])

    return pl.kernel(
        body,
        out_type=jax.ShapeDtypeStruct((_NC, _NACC, _D), jnp.float32),
        mesh=mesh,
        scratch_types=[
            pltpu.VMEM((_CH,), jnp.int32),
            pltpu.VMEM((_CH,), jnp.int32),
            pltpu.VMEM((_CH,), jnp.int32),
            pltpu.VMEM((_CH,), jnp.int32),
            pltpu.VMEM((_CH, _D), jnp.float32),
            pltpu.VMEM_SHARED((_NACC, _D), jnp.float32),
            pltpu.SemaphoreType.DMA,
        ],
    )


@functools.lru_cache(maxsize=None)
def _tc_layer(nk, with_head, bn=1000):
    grid = (_N // bn,)
    dn = (((1,), (1,)), ((), ()))

    def body(*args):
        out_ref = args[-1]
        ps = args[:nk]
        xr = args[nk]
        w0 = nk + 1
        acc = jax.nn.relu(lax.dot_general(xr[...], args[w0 + 2 * nk][...], dn)
                          + args[w0 + 2 * nk + 1][...])
        for i in range(nk):
            agg = ps[i][0] + ps[i][1]
            acc = acc + jax.nn.relu(
                lax.dot_general(agg, args[w0 + 2 * i][...], dn)
                + args[w0 + 2 * i + 1][...])
        h = jax.nn.relu(acc)
        if with_head:
            out_ref[...] = (lax.dot_general(h, args[w0 + 2 * nk + 2][...], dn)
                            + args[w0 + 2 * nk + 3][...])
        else:
            out_ref[...] = h

    p_spec = pl.BlockSpec((_NC, bn, _D), lambda i: (0, i, 0))
    x_spec = pl.BlockSpec((bn, _D), lambda i: (i, 0))
    w_spec = pl.BlockSpec((_D, _D), lambda i: (0, 0))
    b_spec = pl.BlockSpec((1, _D), lambda i: (0, 0))
    n_wb = nk + 1 + (1 if with_head else 0)
    in_specs = ([p_spec] * nk + [x_spec] + [w_spec, b_spec] * n_wb)

    return pl.pallas_call(
        body,
        grid=grid,
        in_specs=in_specs,
        out_specs=pl.BlockSpec((bn, _D), lambda i: (i, 0)),
        out_shape=jax.ShapeDtypeStruct((_N, _D), jnp.float32),
    )


def kernel(x, edge_index, edge_attr, Ws_t0, bs_t0, Wk_t0_k1, bk_t0_k1,
           Ws_t1, bs_t1, Wk_t1_k1, bk_t1_k1, Wk_t1_k2, bk_t1_k2,
           Ws_t2, bs_t2, Wk_t2_k1, bk_t2_k1, Wk_t2_k2, bk_t2_k2,
           Wk_t2_k3, bk_t2_k3, Whead, bhead):
    e = edge_index.shape[1]
    quant = _NC * _NS * _CH
    e_pad = ((e + quant - 1) // quant) * quant
    pad = e_pad - e
    src = jnp.pad(edge_index[0], (0, pad))
    dst = jnp.pad(edge_index[1], (0, pad))
    attr = jnp.pad(edge_attr, (0, pad))
    zeros = jnp.zeros((_NACC, _D), jnp.float32)

    def agg(xt, k):
        return _seg_sum(k, e_pad)(xt, src, dst, attr, zeros)

    def rb(b):
        return b.reshape(1, _D)

    a01 = agg(x, 1)
    h1 = _tc_layer(1, False)(a01, x, Wk_t0_k1, rb(bk_t0_k1),
                             Ws_t0, rb(bs_t0))
    a11 = agg(h1, 1)
    a02 = agg(x, 2)
    h2 = _tc_layer(2, False)(a11, a02, h1, Wk_t1_k1, rb(bk_t1_k1),
                             Wk_t1_k2, rb(bk_t1_k2), Ws_t1, rb(bs_t1))
    a21 = agg(h2, 1)
    a12 = agg(h1, 2)
    a03 = agg(x, 3)
    return _tc_layer(3, True)(a21, a12, a03, h2,
                              Wk_t2_k1, rb(bk_t2_k1), Wk_t2_k2, rb(bk_t2_k2),
                              Wk_t2_k3, rb(bk_t2_k3), Ws_t2, rb(bs_t2),
                              Whead, rb(bhead))
